# Initial kernel scaffold; baseline (speedup 1.0000x reference)
import jax
import jax.numpy as jnp
from jax.experimental import pallas as pl


def kernel(x, edge_index, W1, b1, W2, b2):
    raise NotImplementedError("write your pallas kernel here")


# speedup vs baseline: 20.4064x; 20.4064x over previous
import functools

import jax
import jax.numpy as jnp
from jax import lax
from jax.experimental import pallas as pl
from jax.experimental.pallas import tpu as pltpu
from jax.experimental.pallas import tpu_sc as plsc

NC, NS, LANES = 2, 16, 16
NW = NC * NS
CH = 1024
NRW = CH // 128
RB = 1000


def _mesh():
    return plsc.VectorSubcoreMesh(core_axis_name="c", subcore_axis_name="s")


_SC_PARAMS = pltpu.CompilerParams(use_tc_tiling_on_sc=False)


@functools.partial(jax.jit, static_argnames=("na", "cpt"))
def _sc_degree(dst2, zeros8, ones8, *, na, cpt):
    @functools.partial(
        pl.kernel,
        out_type=jax.ShapeDtypeStruct((NC, na, 8), jnp.float32),
        mesh=_mesh(),
        scratch_types=[
            pltpu.VMEM((NRW, 128), jnp.int32),
            pltpu.VMEM((128, 8), jnp.float32),
            pltpu.VMEM_SHARED((na, 8), jnp.float32),
            pltpu.SemaphoreType.DMA,
        ],
        compiler_params=_SC_PARAMS,
    )
    def k(dst_hbm, zero_hbm, ones_hbm, out_hbm, dst_v, ones_v, acc, sem):
        cid = lax.axis_index("c")
        sid = lax.axis_index("s")
        wid = sid * NC + cid
        rpt = na // NS
        pltpu.sync_copy(zero_hbm.at[pl.ds(sid * rpt, rpt)],
                        acc.at[pl.ds(sid * rpt, rpt)])
        pltpu.sync_copy(ones_hbm, ones_v)
        plsc.subcore_barrier()

        def chunk(kk, carry):
            rbase = wid * (cpt * NRW) + kk * NRW
            pltpu.sync_copy(dst_hbm.at[pl.ds(rbase, NRW)], dst_v)
            hs = [pltpu.async_copy(ones_v, acc.at[dst_v.at[j]], sem, add=True)
                  for j in range(NRW)]
            for h in hs:
                h.wait()
            return carry

        lax.fori_loop(0, cpt, chunk, 0)
        plsc.subcore_barrier()
        pltpu.sync_copy(acc.at[pl.ds(sid * rpt, rpt)],
                        out_hbm.at[cid, pl.ds(sid * rpt, rpt)])

    return k(dst2, zeros8, ones8)


@functools.partial(jax.jit, static_argnames=("na", "cpt"))
def _sc_scatter(g, src2, dst2, zeros16, *, na, cpt):
    @functools.partial(
        pl.kernel,
        out_type=jax.ShapeDtypeStruct((NC, na, LANES), jnp.float32),
        mesh=_mesh(),
        scratch_types=[
            pltpu.VMEM((NRW, 128), jnp.int32),
            pltpu.VMEM((NRW, 128), jnp.int32),
            pltpu.VMEM((CH, LANES), jnp.float32),
            pltpu.VMEM_SHARED((na, LANES), jnp.float32),
            pltpu.SemaphoreType.DMA,
            pltpu.SemaphoreType.DMA,
        ],
        compiler_params=_SC_PARAMS,
    )
    def k(g_hbm, src_hbm, dst_hbm, zero_hbm, out_hbm,
          src_v, dst_v, rows_v, acc, semg, sems):
        cid = lax.axis_index("c")
        sid = lax.axis_index("s")
        wid = sid * NC + cid
        rpt = na // NS
        pltpu.sync_copy(zero_hbm.at[pl.ds(sid * rpt, rpt)],
                        acc.at[pl.ds(sid * rpt, rpt)])
        plsc.subcore_barrier()

        def chunk(kk, carry):
            rbase = wid * (cpt * NRW) + kk * NRW
            pltpu.sync_copy(src_hbm.at[pl.ds(rbase, NRW)], src_v)
            pltpu.sync_copy(dst_hbm.at[pl.ds(rbase, NRW)], dst_v)
            gs = [pltpu.async_copy(g_hbm.at[src_v.at[j]],
                                   rows_v.at[pl.ds(j * 128, 128)], semg)
                  for j in range(NRW)]
            for h in gs:
                h.wait()
            ss = [pltpu.async_copy(rows_v.at[pl.ds(j * 128, 128)],
                                   acc.at[dst_v.at[j]], sems, add=True)
                  for j in range(NRW)]
            for h in ss:
                h.wait()
            return carry

        lax.fori_loop(0, cpt, chunk, 0)
        plsc.subcore_barrier()
        pltpu.sync_copy(acc.at[pl.ds(sid * rpt, rpt)],
                        out_hbm.at[cid, pl.ds(sid * rpt, rpt)])

    return k(g, src2, dst2, zeros16)


def _mm_body(x_ref, w_ref, d0_ref, d1_ref, o_ref):
    dinv = lax.rsqrt(d0_ref[0, 0, :] + d1_ref[0, 0, :] + 1.0)
    h = jnp.dot(x_ref[...], w_ref[...], preferred_element_type=jnp.float32)
    o_ref[...] = h * dinv[:, None]


def _mid_body(p0_ref, p1_ref, g1_ref, d0_ref, d1_ref, b1_ref, w2_ref, o_ref):
    dinv = lax.rsqrt(d0_ref[0, 0, :] + d1_ref[0, 0, :] + 1.0)
    s = (p0_ref[...] + p1_ref[...] + g1_ref[...]) * dinv[:, None]
    out1 = jnp.maximum(s + b1_ref[0, 0, :][None, :], 0.0)
    h2 = jnp.dot(out1, w2_ref[...], preferred_element_type=jnp.float32)
    o_ref[...] = h2 * dinv[:, None]


def _fin_body(nout, q0_ref, q1_ref, g2_ref, d0_ref, d1_ref, b2_ref, o_ref):
    dinv = lax.rsqrt(d0_ref[0, 0, :] + d1_ref[0, 0, :] + 1.0)
    z = (q0_ref[...] + q1_ref[...] + g2_ref[...]) * dinv[:, None]
    z = z + b2_ref[0, 0, :][None, :]
    col = lax.broadcasted_iota(jnp.int32, z.shape, 1)
    zm = jnp.where(col < nout, z, jnp.float32(-1e30))
    m = jnp.max(zm, axis=1, keepdims=True)
    lse = jnp.log(jnp.sum(jnp.exp(zm - m), axis=1, keepdims=True)) + m
    o_ref[...] = (z - lse)[:, :nout]


def _row_spec():
    return pl.BlockSpec((1, 1, RB), lambda i: (i, 0, 0))


def _tc_mm(x, w1, d0, d1):
    n, f = x.shape
    hid = w1.shape[1]
    return pl.pallas_call(
        _mm_body,
        grid=(n // RB,),
        in_specs=[
            pl.BlockSpec((RB, f), lambda i: (i, 0)),
            pl.BlockSpec((f, hid), lambda i: (0, 0)),
            _row_spec(), _row_spec(),
        ],
        out_specs=pl.BlockSpec((RB, hid), lambda i: (i, 0)),
        out_shape=jax.ShapeDtypeStruct((n, hid), jnp.float32),
    )(x, w1, d0, d1)


def _tc_mid(p0, p1, g1, d0, d1, b1r, w2p):
    n, hid = g1.shape
    blk = pl.BlockSpec((RB, hid), lambda i: (i, 0))
    return pl.pallas_call(
        _mid_body,
        grid=(n // RB,),
        in_specs=[
            blk, blk, blk, _row_spec(), _row_spec(),
            pl.BlockSpec((1, 1, hid), lambda i: (0, 0, 0)),
            pl.BlockSpec((hid, hid), lambda i: (0, 0)),
        ],
        out_specs=blk,
        out_shape=jax.ShapeDtypeStruct((n, hid), jnp.float32),
    )(p0, p1, g1, d0, d1, b1r, w2p)


def _tc_final(q0, q1, g2, d0, d1, b2r, nout):
    n, hid = g2.shape
    blk = pl.BlockSpec((RB, hid), lambda i: (i, 0))
    return pl.pallas_call(
        functools.partial(_fin_body, nout),
        grid=(n // RB,),
        in_specs=[
            blk, blk, blk, _row_spec(), _row_spec(),
            pl.BlockSpec((1, 1, hid), lambda i: (0, 0, 0)),
        ],
        out_specs=pl.BlockSpec((RB, nout), lambda i: (i, 0)),
        out_shape=jax.ShapeDtypeStruct((n, nout), jnp.float32),
    )(q0, q1, g2, d0, d1, b2r)


def kernel(x, edge_index, W1, b1, W2, b2):
    n, _ = x.shape
    e = edge_index.shape[1]
    hid = W1.shape[1]
    nout = W2.shape[1]
    assert hid == LANES and n % RB == 0

    cpt = -(-e // (NW * CH))
    ep = NW * CH * cpt
    na = -(-(n + 1) // (NS * 8)) * (NS * 8)

    src = edge_index[0]
    dst = edge_index[1]
    pad = ep - e
    src2 = jnp.concatenate([src, jnp.zeros((pad,), jnp.int32)]).reshape(-1, 128)
    dst2 = jnp.concatenate([dst, jnp.full((pad,), n, jnp.int32)]).reshape(-1, 128)
    zeros16 = jnp.zeros((na, LANES), jnp.float32)
    zeros8 = jnp.zeros((na, 8), jnp.float32)
    ones8 = jnp.zeros((128, 8), jnp.float32).at[:, 0].set(1.0)

    degp = _sc_degree(dst2, zeros8, ones8, na=na, cpt=cpt)
    d0 = degp[0, :n, 0].reshape(-1, 1, RB)
    d1 = degp[1, :n, 0].reshape(-1, 1, RB)

    g1 = _tc_mm(x, W1, d0, d1)

    p = _sc_scatter(g1, src2, dst2, zeros16, na=na, cpt=cpt)

    w2p = jnp.pad(W2, ((0, 0), (0, hid - nout)))
    b1r = b1.reshape(1, 1, hid)
    g2 = _tc_mid(p[0, :n], p[1, :n], g1, d0, d1, b1r, w2p)

    q = _sc_scatter(g2, src2, dst2, zeros16, na=na, cpt=cpt)

    b2r = jnp.pad(b2, (0, hid - nout)).reshape(1, 1, hid)
    return _tc_final(q[0, :n], q[1, :n], g2, d0, d1, b2r, nout)


# speedup vs baseline: 32.6789x; 1.6014x over previous
import functools

import jax
import jax.numpy as jnp
from jax import lax
from jax.experimental import pallas as pl
from jax.experimental.pallas import tpu as pltpu
from jax.experimental.pallas import tpu_sc as plsc

NC, NS, LANES = 2, 16, 16
NW = NC * NS
CH = 1024
NRW = CH // 128
RB = 1024
PK = LANES * 8


def _mesh():
    return plsc.VectorSubcoreMesh(core_axis_name="c", subcore_axis_name="s")


_SC_PARAMS = pltpu.CompilerParams(use_tc_tiling_on_sc=False)


@functools.partial(jax.jit, static_argnames=("na", "cpt"))
def _sc_degree(dst2, zeros16, ones16, *, na, cpt):
    @functools.partial(
        pl.kernel,
        out_type=(jax.ShapeDtypeStruct((na, LANES), jnp.float32),
                  jax.ShapeDtypeStruct((na, LANES), jnp.float32)),
        mesh=_mesh(),
        scratch_types=[
            pltpu.VMEM((NRW, 128), jnp.int32),
            pltpu.VMEM((128, LANES), jnp.float32),
            pltpu.VMEM_SHARED((na, LANES), jnp.float32),
            pltpu.SemaphoreType.DMA,
        ],
        compiler_params=_SC_PARAMS,
    )
    def k(dst_hbm, zero_hbm, ones_hbm, out0, out1, dst_v, ones_v, acc, sem):
        cid = lax.axis_index("c")
        sid = lax.axis_index("s")
        wid = sid * NC + cid
        zpt = na // NS
        pltpu.sync_copy(zero_hbm.at[pl.ds(sid * zpt, zpt)],
                        acc.at[pl.ds(sid * zpt, zpt)])
        pltpu.sync_copy(ones_hbm, ones_v)
        plsc.subcore_barrier()

        def chunk(kk, carry):
            rbase = wid * (cpt * NRW) + kk * NRW
            pltpu.sync_copy(dst_hbm.at[pl.ds(rbase, NRW)], dst_v)
            hs = [pltpu.async_copy(ones_v, acc.at[dst_v.at[j]], sem, add=True)
                  for j in range(NRW)]
            for h in hs:
                h.wait()
            return carry

        lax.fori_loop(0, cpt, chunk, 0)
        plsc.subcore_barrier()

        @pl.when(cid == 0)
        def _():
            pltpu.sync_copy(acc.at[pl.ds(sid * zpt, zpt)],
                            out0.at[pl.ds(sid * zpt, zpt)])

        @pl.when(cid == 1)
        def _():
            pltpu.sync_copy(acc.at[pl.ds(sid * zpt, zpt)],
                            out1.at[pl.ds(sid * zpt, zpt)])

    return k(dst2, zeros16, ones16)


@functools.partial(jax.jit, static_argnames=("na", "cpt"))
def _sc_scatter(g, src2, dst2, zeros16, *, na, cpt):
    @functools.partial(
        pl.kernel,
        out_type=(jax.ShapeDtypeStruct((na, LANES), jnp.float32),
                  jax.ShapeDtypeStruct((na, LANES), jnp.float32)),
        mesh=_mesh(),
        scratch_types=[
            pltpu.VMEM((NRW, 128), jnp.int32),
            pltpu.VMEM((NRW, 128), jnp.int32),
            pltpu.VMEM((CH, LANES), jnp.float32),
            pltpu.VMEM_SHARED((na, LANES), jnp.float32),
            pltpu.SemaphoreType.DMA,
            pltpu.SemaphoreType.DMA,
        ],
        compiler_params=_SC_PARAMS,
    )
    def k(g_hbm, src_hbm, dst_hbm, zero_hbm, out0, out1,
          src_v, dst_v, rows_v, acc, semg, sems):
        cid = lax.axis_index("c")
        sid = lax.axis_index("s")
        wid = sid * NC + cid
        zpt = na // NS
        pltpu.sync_copy(zero_hbm.at[pl.ds(sid * zpt, zpt)],
                        acc.at[pl.ds(sid * zpt, zpt)])
        plsc.subcore_barrier()

        def chunk(kk, carry):
            rbase = wid * (cpt * NRW) + kk * NRW
            pltpu.sync_copy(src_hbm.at[pl.ds(rbase, NRW)], src_v)
            pltpu.sync_copy(dst_hbm.at[pl.ds(rbase, NRW)], dst_v)
            gs = [pltpu.async_copy(g_hbm.at[src_v.at[j]],
                                   rows_v.at[pl.ds(j * 128, 128)], semg)
                  for j in range(NRW)]
            for h in gs:
                h.wait()
            ss = [pltpu.async_copy(rows_v.at[pl.ds(j * 128, 128)],
                                   acc.at[dst_v.at[j]], sems, add=True)
                  for j in range(NRW)]
            for h in ss:
                h.wait()
            return carry

        lax.fori_loop(0, cpt, chunk, 0)
        plsc.subcore_barrier()

        @pl.when(cid == 0)
        def _():
            pltpu.sync_copy(acc.at[pl.ds(sid * zpt, zpt)],
                            out0.at[pl.ds(sid * zpt, zpt)])

        @pl.when(cid == 1)
        def _():
            pltpu.sync_copy(acc.at[pl.ds(sid * zpt, zpt)],
                            out1.at[pl.ds(sid * zpt, zpt)])

    return k(g, src2, dst2, zeros16)


def _mm_body(xt_ref, w1t_ref, h_ref):
    h_ref[...] = lax.dot_general(xt_ref[...], w1t_ref[...],
                                 (((0,), (1,)), ((), ())),
                                 preferred_element_type=jnp.float32)


def _scale_body(h_ref, d0_ref, d1_ref, g_ref):
    dinv = lax.rsqrt(d0_ref[...] + d1_ref[...] + 1.0)
    g_ref[...] = h_ref[...] * dinv


def _mid_body(p0_ref, p1_ref, g1_ref, d0_ref, d1_ref, b1_ref, w2_ref, o_ref):
    dinv = lax.rsqrt(d0_ref[...] + d1_ref[...] + 1.0)
    s = (p0_ref[...] + p1_ref[...] + g1_ref[...]) * dinv
    out1 = jnp.maximum(s + b1_ref[0, :][None, :], 0.0)
    h2 = jnp.dot(out1, w2_ref[...], preferred_element_type=jnp.float32)
    o_ref[...] = h2 * dinv


def _fin_body(nout, q0_ref, q1_ref, g2_ref, d0_ref, d1_ref, b2_ref, ones_ref,
              o_ref):
    dinv = lax.rsqrt(d0_ref[...] + d1_ref[...] + 1.0)
    z = (q0_ref[...] + q1_ref[...] + g2_ref[...]) * dinv + b2_ref[0, :][None, :]
    feat = lax.broadcasted_iota(jnp.int32, z.shape, 1) % LANES
    e = jnp.where(feat < nout, jnp.exp(z), 0.0)
    s = jnp.dot(e, ones_ref[...], preferred_element_type=jnp.float32)
    o_ref[...] = z - jnp.log(s)


def _pk():
    return pl.BlockSpec((RB // 8, PK), lambda i: (i, 0))


def _row():
    return pl.BlockSpec((1, PK), lambda i: (0, 0))


def _sq():
    return pl.BlockSpec((PK, PK), lambda i: (0, 0))


def _tc_mm(xt, w1t, na):
    f = xt.shape[0]
    return pl.pallas_call(
        _mm_body,
        grid=(-(-na // RB),),
        in_specs=[
            pl.BlockSpec((f, RB), lambda i: (0, i)),
            pl.BlockSpec((LANES, f), lambda i: (0, 0)),
        ],
        out_specs=pl.BlockSpec((RB, LANES), lambda i: (i, 0)),
        out_shape=jax.ShapeDtypeStruct((na, LANES), jnp.float32),
    )(xt, w1t)


def _tc_scale(hpk, d0, d1, nr):
    return pl.pallas_call(
        _scale_body,
        grid=(-(-nr // (RB // 8)),),
        in_specs=[_pk(), _pk(), _pk()],
        out_specs=_pk(),
        out_shape=jax.ShapeDtypeStruct((nr, PK), jnp.float32),
    )(hpk, d0, d1)


def _tc_mid(p0, p1, g1, d0, d1, b1r, w2big, nr):
    return pl.pallas_call(
        _mid_body,
        grid=(-(-nr // (RB // 8)),),
        in_specs=[_pk(), _pk(), _pk(), _pk(), _pk(), _row(), _sq()],
        out_specs=_pk(),
        out_shape=jax.ShapeDtypeStruct((nr, PK), jnp.float32),
    )(p0, p1, g1, d0, d1, b1r, w2big)


def _tc_final(q0, q1, g2, d0, d1, b2r, onesbig, nr, nout):
    return pl.pallas_call(
        functools.partial(_fin_body, nout),
        grid=(-(-nr // (RB // 8)),),
        in_specs=[_pk(), _pk(), _pk(), _pk(), _pk(), _row(), _sq()],
        out_specs=_pk(),
        out_shape=jax.ShapeDtypeStruct((nr, PK), jnp.float32),
    )(q0, q1, g2, d0, d1, b2r, onesbig)


def kernel(x, edge_index, W1, b1, W2, b2):
    n, _ = x.shape
    e = edge_index.shape[1]
    hid = W1.shape[1]
    nout = W2.shape[1]
    assert hid == LANES

    cpt = -(-e // (NW * CH))
    ep = NW * CH * cpt
    na = -(-(n + 1) // (NS * 8)) * (NS * 8)
    nr = na * LANES // PK

    src = edge_index[0]
    dst = edge_index[1]
    pad = ep - e
    src2 = jnp.concatenate([src, jnp.zeros((pad,), jnp.int32)]).reshape(-1, 128)
    dst2 = jnp.concatenate([dst, jnp.full((pad,), n, jnp.int32)]).reshape(-1, 128)
    zeros16 = jnp.zeros((na, LANES), jnp.float32)
    ones16 = jnp.ones((128, LANES), jnp.float32)

    deg0, deg1 = _sc_degree(dst2, zeros16, ones16, na=na, cpt=cpt)
    d0 = deg0.reshape(nr, PK)
    d1 = deg1.reshape(nr, PK)

    h1 = _tc_mm(x.T, W1.T, na)
    g1 = _tc_scale(h1.reshape(nr, PK), d0, d1, nr)

    p0, p1 = _sc_scatter(g1.reshape(na, LANES), src2, dst2, zeros16,
                         na=na, cpt=cpt)

    w2big = jnp.kron(jnp.eye(8, dtype=jnp.float32),
                     jnp.pad(W2, ((0, 0), (0, hid - nout))))
    b1r = jnp.tile(b1, 8).reshape(1, PK)
    g2 = _tc_mid(p0.reshape(nr, PK), p1.reshape(nr, PK), g1, d0, d1,
                 b1r, w2big, nr)

    q0, q1 = _sc_scatter(g2.reshape(na, LANES), src2, dst2, zeros16,
                         na=na, cpt=cpt)

    onesbig = jnp.kron(jnp.eye(8, dtype=jnp.float32),
                       jnp.ones((LANES, LANES), jnp.float32))
    b2r = jnp.tile(jnp.pad(b2, (0, hid - nout)), 8).reshape(1, PK)
    outpk = _tc_final(q0.reshape(nr, PK), q1.reshape(nr, PK), g2, d0, d1,
                      b2r, onesbig, nr, nout)
    return outpk.reshape(na, LANES)[:n, :nout]


# speedup vs baseline: 37.9506x; 1.1613x over previous
import functools

import jax
import jax.numpy as jnp
from jax import lax
from jax.experimental import pallas as pl
from jax.experimental.pallas import tpu as pltpu
from jax.experimental.pallas import tpu_sc as plsc

NC, NS, LANES = 2, 16, 16
NW = NC * NS
CH = 1024
NRW = CH // 128
RB = 1024
PK = LANES * 8


def _mesh():
    return plsc.VectorSubcoreMesh(core_axis_name="c", subcore_axis_name="s")


_SC_PARAMS = pltpu.CompilerParams(use_tc_tiling_on_sc=False)


@functools.partial(jax.jit, static_argnames=("na", "cpt"))
def _sc_degree(dst2, zeros16, ones16, *, na, cpt):
    @functools.partial(
        pl.kernel,
        out_type=(jax.ShapeDtypeStruct((na, LANES), jnp.float32),
                  jax.ShapeDtypeStruct((na, LANES), jnp.float32)),
        mesh=_mesh(),
        scratch_types=[
            pltpu.VMEM((NRW, 128), jnp.int32),
            pltpu.VMEM((128, LANES), jnp.float32),
            pltpu.VMEM_SHARED((na, LANES), jnp.float32),
            pltpu.SemaphoreType.DMA,
        ],
        compiler_params=_SC_PARAMS,
    )
    def k(dst_hbm, zero_hbm, ones_hbm, out0, out1, dst_v, ones_v, acc, sem):
        cid = lax.axis_index("c")
        sid = lax.axis_index("s")
        wid = sid * NC + cid
        zpt = na // NS
        pltpu.sync_copy(zero_hbm.at[pl.ds(sid * zpt, zpt)],
                        acc.at[pl.ds(sid * zpt, zpt)])
        pltpu.sync_copy(ones_hbm, ones_v)
        plsc.subcore_barrier()

        def chunk(kk, carry):
            rbase = wid * (cpt * NRW) + kk * NRW
            pltpu.sync_copy(dst_hbm.at[pl.ds(rbase, NRW)], dst_v)
            hs = [pltpu.async_copy(ones_v, acc.at[dst_v.at[j]], sem, add=True)
                  for j in range(NRW)]
            for h in hs:
                h.wait()
            return carry

        lax.fori_loop(0, cpt, chunk, 0)
        plsc.subcore_barrier()

        @pl.when(cid == 0)
        def _():
            pltpu.sync_copy(acc.at[pl.ds(sid * zpt, zpt)],
                            out0.at[pl.ds(sid * zpt, zpt)])

        @pl.when(cid == 1)
        def _():
            pltpu.sync_copy(acc.at[pl.ds(sid * zpt, zpt)],
                            out1.at[pl.ds(sid * zpt, zpt)])

    return k(dst2, zeros16, ones16)


@functools.partial(jax.jit, static_argnames=("na", "cpt"))
def _sc_scatter(g, src2, dst2, zeros16, *, na, cpt):
    @functools.partial(
        pl.kernel,
        out_type=(jax.ShapeDtypeStruct((na, LANES), jnp.float32),
                  jax.ShapeDtypeStruct((na, LANES), jnp.float32)),
        mesh=_mesh(),
        scratch_types=[
            pltpu.VMEM((2, NRW, 128), jnp.int32),
            pltpu.VMEM((2, NRW, 128), jnp.int32),
            pltpu.VMEM((2, CH, LANES), jnp.float32),
            pltpu.VMEM_SHARED((na, LANES), jnp.float32),
            pltpu.SemaphoreType.DMA,
            pltpu.SemaphoreType.DMA,
            pltpu.SemaphoreType.DMA,
            pltpu.SemaphoreType.DMA,
        ],
        compiler_params=_SC_PARAMS,
    )
    def k(g_hbm, src_hbm, dst_hbm, zero_hbm, out0, out1,
          src_v, dst_v, rows_v, acc, semg0, semg1, sems0, sems1):
        cid = lax.axis_index("c")
        sid = lax.axis_index("s")
        wid = sid * NC + cid
        zpt = na // NS
        pltpu.sync_copy(zero_hbm.at[pl.ds(sid * zpt, zpt)],
                        acc.at[pl.ds(sid * zpt, zpt)])
        plsc.subcore_barrier()

        semg = (semg0, semg1)
        sems = (sems0, sems1)
        pend = [None, None]
        for kk in range(cpt):
            b = kk % 2
            if pend[b] is not None:
                for h in pend[b]:
                    h.wait()
                pend[b] = None
            rbase = wid * (cpt * NRW) + kk * NRW
            pltpu.sync_copy(src_hbm.at[pl.ds(rbase, NRW)], src_v.at[b])
            pltpu.sync_copy(dst_hbm.at[pl.ds(rbase, NRW)], dst_v.at[b])
            gs = [pltpu.async_copy(g_hbm.at[src_v.at[b, j]],
                                   rows_v.at[b, pl.ds(j * 128, 128)], semg[b])
                  for j in range(NRW)]
            for h in gs:
                h.wait()
            pend[b] = [pltpu.async_copy(rows_v.at[b, pl.ds(j * 128, 128)],
                                        acc.at[dst_v.at[b, j]], sems[b],
                                        add=True)
                       for j in range(NRW)]
        for hs in pend:
            if hs is not None:
                for h in hs:
                    h.wait()
        plsc.subcore_barrier()

        @pl.when(cid == 0)
        def _():
            pltpu.sync_copy(acc.at[pl.ds(sid * zpt, zpt)],
                            out0.at[pl.ds(sid * zpt, zpt)])

        @pl.when(cid == 1)
        def _():
            pltpu.sync_copy(acc.at[pl.ds(sid * zpt, zpt)],
                            out1.at[pl.ds(sid * zpt, zpt)])

    return k(g, src2, dst2, zeros16)


def _mm_body(xt_ref, w1t_ref, h_ref):
    h_ref[...] = lax.dot_general(xt_ref[...], w1t_ref[...],
                                 (((0,), (1,)), ((), ())),
                                 preferred_element_type=jnp.float32)


def _scale_body(h_ref, d0_ref, d1_ref, g_ref):
    dinv = lax.rsqrt(d0_ref[...] + d1_ref[...] + 1.0)
    g_ref[...] = h_ref[...] * dinv


def _mid_body(p0_ref, p1_ref, g1_ref, d0_ref, d1_ref, b1_ref, w2_ref, o_ref):
    dinv = lax.rsqrt(d0_ref[...] + d1_ref[...] + 1.0)
    s = (p0_ref[...] + p1_ref[...] + g1_ref[...]) * dinv
    out1 = jnp.maximum(s + b1_ref[0, :][None, :], 0.0)
    h2 = jnp.dot(out1, w2_ref[...], preferred_element_type=jnp.float32)
    o_ref[...] = h2 * dinv


def _fin_body(nout, q0_ref, q1_ref, g2_ref, d0_ref, d1_ref, b2_ref, ones_ref,
              o_ref):
    dinv = lax.rsqrt(d0_ref[...] + d1_ref[...] + 1.0)
    z = (q0_ref[...] + q1_ref[...] + g2_ref[...]) * dinv + b2_ref[0, :][None, :]
    feat = lax.broadcasted_iota(jnp.int32, z.shape, 1) % LANES
    e = jnp.where(feat < nout, jnp.exp(z), 0.0)
    s = jnp.dot(e, ones_ref[...], preferred_element_type=jnp.float32)
    o_ref[...] = z - jnp.log(s)


PKR = 448


def _pk():
    return pl.BlockSpec((PKR, PK), lambda i: (i, 0))


def _row():
    return pl.BlockSpec((1, PK), lambda i: (0, 0))


def _sq():
    return pl.BlockSpec((PK, PK), lambda i: (0, 0))


def _tc_mm(xt, w1t, na):
    f = xt.shape[0]
    return pl.pallas_call(
        _mm_body,
        grid=(-(-na // RB),),
        in_specs=[
            pl.BlockSpec((f, RB), lambda i: (0, i)),
            pl.BlockSpec((LANES, f), lambda i: (0, 0)),
        ],
        out_specs=pl.BlockSpec((RB, LANES), lambda i: (i, 0)),
        out_shape=jax.ShapeDtypeStruct((na, LANES), jnp.float32),
    )(xt, w1t)


def _tc_scale(hpk, d0, d1, nr):
    return pl.pallas_call(
        _scale_body,
        grid=(-(-nr // PKR),),
        in_specs=[_pk(), _pk(), _pk()],
        out_specs=_pk(),
        out_shape=jax.ShapeDtypeStruct((nr, PK), jnp.float32),
    )(hpk, d0, d1)


def _tc_mid(p0, p1, g1, d0, d1, b1r, w2big, nr):
    return pl.pallas_call(
        _mid_body,
        grid=(-(-nr // PKR),),
        in_specs=[_pk(), _pk(), _pk(), _pk(), _pk(), _row(), _sq()],
        out_specs=_pk(),
        out_shape=jax.ShapeDtypeStruct((nr, PK), jnp.float32),
    )(p0, p1, g1, d0, d1, b1r, w2big)


def _tc_final(q0, q1, g2, d0, d1, b2r, onesbig, nr, nout):
    return pl.pallas_call(
        functools.partial(_fin_body, nout),
        grid=(-(-nr // PKR),),
        in_specs=[_pk(), _pk(), _pk(), _pk(), _pk(), _row(), _sq()],
        out_specs=_pk(),
        out_shape=jax.ShapeDtypeStruct((nr, PK), jnp.float32),
    )(q0, q1, g2, d0, d1, b2r, onesbig)


def kernel(x, edge_index, W1, b1, W2, b2):
    n, _ = x.shape
    e = edge_index.shape[1]
    hid = W1.shape[1]
    nout = W2.shape[1]
    assert hid == LANES

    cpt = -(-e // (NW * CH))
    ept = cpt * CH
    na = -(-(n + 128) // 128) * 128
    nr = na * LANES // PK

    e2 = -(-e // NW) * NW
    rpt = e2 // NW
    pad2 = ept - rpt
    src = edge_index[0]
    dst = edge_index[1]
    if e2 > e:
        src = jnp.concatenate([src, jnp.zeros((e2 - e,), jnp.int32)])
        dst = jnp.concatenate(
            [dst, n + (jnp.arange(e2 - e, dtype=jnp.int32) % 128)])
    dum = n + (jnp.arange(NW * pad2, dtype=jnp.int32) % 128)
    src2 = jnp.concatenate(
        [src.reshape(NW, rpt), jnp.zeros((NW, pad2), jnp.int32)],
        axis=1).reshape(-1, 128)
    dst2 = jnp.concatenate(
        [dst.reshape(NW, rpt), dum.reshape(NW, pad2)],
        axis=1).reshape(-1, 128)
    zeros16 = jnp.zeros((na, LANES), jnp.float32)
    ones16 = jnp.ones((128, LANES), jnp.float32)

    deg0, deg1 = _sc_degree(dst2, zeros16, ones16, na=na, cpt=cpt)
    d0 = deg0.reshape(nr, PK)
    d1 = deg1.reshape(nr, PK)

    h1 = _tc_mm(x.T, W1.T, na)
    g1 = _tc_scale(h1.reshape(nr, PK), d0, d1, nr)

    p0, p1 = _sc_scatter(g1.reshape(na, LANES), src2, dst2, zeros16,
                         na=na, cpt=cpt)

    w2big = jnp.kron(jnp.eye(8, dtype=jnp.float32),
                     jnp.pad(W2, ((0, 0), (0, hid - nout))))
    b1r = jnp.tile(b1, 8).reshape(1, PK)
    g2 = _tc_mid(p0.reshape(nr, PK), p1.reshape(nr, PK), g1, d0, d1,
                 b1r, w2big, nr)

    q0, q1 = _sc_scatter(g2.reshape(na, LANES), src2, dst2, zeros16,
                         na=na, cpt=cpt)

    onesbig = jnp.kron(jnp.eye(8, dtype=jnp.float32),
                       jnp.ones((LANES, LANES), jnp.float32))
    b2r = jnp.tile(jnp.pad(b2, (0, hid - nout)), 8).reshape(1, PK)
    outpk = _tc_final(q0.reshape(nr, PK), q1.reshape(nr, PK), g2, d0, d1,
                      b2r, onesbig, nr, nout)
    return outpk.reshape(na, LANES)[:n, :nout]


# speedup vs baseline: 50.8505x; 1.3399x over previous
import functools

import jax
import jax.numpy as jnp
from jax import lax
from jax.experimental import pallas as pl
from jax.experimental.pallas import tpu as pltpu
from jax.experimental.pallas import tpu_sc as plsc

NC, NS, LANES = 2, 16, 16
NW = NC * NS
CH = 512
NRW = CH // 128
RB = 1024
PK = LANES * 8


def _mesh():
    return plsc.VectorSubcoreMesh(core_axis_name="c", subcore_axis_name="s")


_SC_PARAMS = pltpu.CompilerParams(use_tc_tiling_on_sc=False)


@functools.partial(jax.jit, static_argnames=("na", "cpt"))
def _sc_degree(dst2, zeros16, ones16, *, na, cpt):
    @functools.partial(
        pl.kernel,
        out_type=(jax.ShapeDtypeStruct((na, LANES), jnp.float32),
                  jax.ShapeDtypeStruct((na, LANES), jnp.float32)),
        mesh=_mesh(),
        scratch_types=[
            pltpu.VMEM((NRW, 128), jnp.int32),
            pltpu.VMEM((128, LANES), jnp.float32),
            pltpu.VMEM_SHARED((na, LANES), jnp.float32),
            pltpu.SemaphoreType.DMA,
        ],
        compiler_params=_SC_PARAMS,
    )
    def k(dst_hbm, zero_hbm, ones_hbm, out0, out1, dst_v, ones_v, acc, sem):
        cid = lax.axis_index("c")
        sid = lax.axis_index("s")
        wid = sid * NC + cid
        zpt = na // NS
        pltpu.sync_copy(zero_hbm.at[pl.ds(sid * zpt, zpt)],
                        acc.at[pl.ds(sid * zpt, zpt)])
        pltpu.sync_copy(ones_hbm, ones_v)
        plsc.subcore_barrier()

        def chunk(kk, carry):
            rbase = wid * (cpt * NRW) + kk * NRW
            pltpu.sync_copy(dst_hbm.at[pl.ds(rbase, NRW)], dst_v)
            hs = [pltpu.async_copy(ones_v, acc.at[dst_v.at[j]], sem, add=True)
                  for j in range(NRW)]
            for h in hs:
                h.wait()
            return carry

        lax.fori_loop(0, cpt, chunk, 0)
        plsc.subcore_barrier()

        @pl.when(cid == 0)
        def _():
            pltpu.sync_copy(acc.at[pl.ds(sid * zpt, zpt)],
                            out0.at[pl.ds(sid * zpt, zpt)])

        @pl.when(cid == 1)
        def _():
            pltpu.sync_copy(acc.at[pl.ds(sid * zpt, zpt)],
                            out1.at[pl.ds(sid * zpt, zpt)])

    return k(dst2, zeros16, ones16)


@functools.partial(jax.jit, static_argnames=("na", "cpt"))
def _sc_scatter(g, src2, dst2, zeros16, *, na, cpt):
    @functools.partial(
        pl.kernel,
        out_type=(jax.ShapeDtypeStruct((na, LANES), jnp.float32),
                  jax.ShapeDtypeStruct((na, LANES), jnp.float32)),
        mesh=_mesh(),
        scratch_types=[
            pltpu.VMEM((2, NRW, 128), jnp.int32),
            pltpu.VMEM((2, NRW, 128), jnp.int32),
            pltpu.VMEM((2, CH, LANES), jnp.float32),
            pltpu.VMEM_SHARED((na, LANES), jnp.float32),
            pltpu.VMEM_SHARED((na, LANES), jnp.float32),
            pltpu.SemaphoreType.DMA,
            pltpu.SemaphoreType.DMA,
            pltpu.SemaphoreType.DMA,
            pltpu.SemaphoreType.DMA,
        ],
        compiler_params=_SC_PARAMS,
    )
    def k(g_hbm, src_hbm, dst_hbm, zero_hbm, out0, out1,
          src_v, dst_v, rows_v, acc, gtab, semg0, semg1, sems0, sems1):
        cid = lax.axis_index("c")
        sid = lax.axis_index("s")
        wid = sid * NC + cid
        zpt = na // NS
        pltpu.sync_copy(zero_hbm.at[pl.ds(sid * zpt, zpt)],
                        acc.at[pl.ds(sid * zpt, zpt)])
        pltpu.sync_copy(g_hbm.at[pl.ds(sid * zpt, zpt)],
                        gtab.at[pl.ds(sid * zpt, zpt)])
        plsc.subcore_barrier()

        semg = (semg0, semg1)
        sems = (sems0, sems1)
        pend = [None, None]
        for kk in range(cpt):
            b = kk % 2
            if pend[b] is not None:
                for h in pend[b]:
                    h.wait()
                pend[b] = None
            rbase = wid * (cpt * NRW) + kk * NRW
            pltpu.sync_copy(src_hbm.at[pl.ds(rbase, NRW)], src_v.at[b])
            pltpu.sync_copy(dst_hbm.at[pl.ds(rbase, NRW)], dst_v.at[b])
            gs = [pltpu.async_copy(gtab.at[src_v.at[b, j]],
                                   rows_v.at[b, pl.ds(j * 128, 128)], semg[b])
                  for j in range(NRW)]
            for h in gs:
                h.wait()
            pend[b] = [pltpu.async_copy(rows_v.at[b, pl.ds(j * 128, 128)],
                                        acc.at[dst_v.at[b, j]], sems[b],
                                        add=True)
                       for j in range(NRW)]
        for hs in pend:
            if hs is not None:
                for h in hs:
                    h.wait()
        plsc.subcore_barrier()

        @pl.when(cid == 0)
        def _():
            pltpu.sync_copy(acc.at[pl.ds(sid * zpt, zpt)],
                            out0.at[pl.ds(sid * zpt, zpt)])

        @pl.when(cid == 1)
        def _():
            pltpu.sync_copy(acc.at[pl.ds(sid * zpt, zpt)],
                            out1.at[pl.ds(sid * zpt, zpt)])

    return k(g, src2, dst2, zeros16)


def _mm_body(xt_ref, w1t_ref, h_ref):
    h_ref[...] = lax.dot_general(xt_ref[...], w1t_ref[...],
                                 (((0,), (1,)), ((), ())),
                                 preferred_element_type=jnp.float32)


def _scale_body(h_ref, d0_ref, d1_ref, g_ref):
    dinv = lax.rsqrt(d0_ref[...] + d1_ref[...] + 1.0)
    g_ref[...] = h_ref[...] * dinv


def _mid_body(p0_ref, p1_ref, g1_ref, d0_ref, d1_ref, b1_ref, w2_ref, o_ref):
    dinv = lax.rsqrt(d0_ref[...] + d1_ref[...] + 1.0)
    s = (p0_ref[...] + p1_ref[...] + g1_ref[...]) * dinv
    out1 = jnp.maximum(s + b1_ref[0, :][None, :], 0.0)
    h2 = jnp.dot(out1, w2_ref[...], preferred_element_type=jnp.float32)
    o_ref[...] = h2 * dinv


def _fin_body(nout, q0_ref, q1_ref, g2_ref, d0_ref, d1_ref, b2_ref, ones_ref,
              o_ref):
    dinv = lax.rsqrt(d0_ref[...] + d1_ref[...] + 1.0)
    z = (q0_ref[...] + q1_ref[...] + g2_ref[...]) * dinv + b2_ref[0, :][None, :]
    feat = lax.broadcasted_iota(jnp.int32, z.shape, 1) % LANES
    e = jnp.where(feat < nout, jnp.exp(z), 0.0)
    s = jnp.dot(e, ones_ref[...], preferred_element_type=jnp.float32)
    o_ref[...] = z - jnp.log(s)


PKR = 448


def _pk():
    return pl.BlockSpec((PKR, PK), lambda i: (i, 0))


def _row():
    return pl.BlockSpec((1, PK), lambda i: (0, 0))


def _sq():
    return pl.BlockSpec((PK, PK), lambda i: (0, 0))


def _tc_mm(xt, w1t, na):
    f = xt.shape[0]
    return pl.pallas_call(
        _mm_body,
        grid=(-(-na // RB),),
        in_specs=[
            pl.BlockSpec((f, RB), lambda i: (0, i)),
            pl.BlockSpec((LANES, f), lambda i: (0, 0)),
        ],
        out_specs=pl.BlockSpec((RB, LANES), lambda i: (i, 0)),
        out_shape=jax.ShapeDtypeStruct((na, LANES), jnp.float32),
    )(xt, w1t)


def _tc_scale(hpk, d0, d1, nr):
    return pl.pallas_call(
        _scale_body,
        grid=(-(-nr // PKR),),
        in_specs=[_pk(), _pk(), _pk()],
        out_specs=_pk(),
        out_shape=jax.ShapeDtypeStruct((nr, PK), jnp.float32),
    )(hpk, d0, d1)


def _tc_mid(p0, p1, g1, d0, d1, b1r, w2big, nr):
    return pl.pallas_call(
        _mid_body,
        grid=(-(-nr // PKR),),
        in_specs=[_pk(), _pk(), _pk(), _pk(), _pk(), _row(), _sq()],
        out_specs=_pk(),
        out_shape=jax.ShapeDtypeStruct((nr, PK), jnp.float32),
    )(p0, p1, g1, d0, d1, b1r, w2big)


def _tc_final(q0, q1, g2, d0, d1, b2r, onesbig, nr, nout):
    return pl.pallas_call(
        functools.partial(_fin_body, nout),
        grid=(-(-nr // PKR),),
        in_specs=[_pk(), _pk(), _pk(), _pk(), _pk(), _row(), _sq()],
        out_specs=_pk(),
        out_shape=jax.ShapeDtypeStruct((nr, PK), jnp.float32),
    )(q0, q1, g2, d0, d1, b2r, onesbig)


def kernel(x, edge_index, W1, b1, W2, b2):
    n, _ = x.shape
    e = edge_index.shape[1]
    hid = W1.shape[1]
    nout = W2.shape[1]
    assert hid == LANES

    cpt = -(-e // (NW * CH))
    ept = cpt * CH
    na = -(-(n + 128) // 128) * 128
    nr = na * LANES // PK

    e2 = -(-e // NW) * NW
    rpt = e2 // NW
    pad2 = ept - rpt
    src = edge_index[0]
    dst = edge_index[1]
    if e2 > e:
        src = jnp.concatenate([src, jnp.zeros((e2 - e,), jnp.int32)])
        dst = jnp.concatenate(
            [dst, n + (jnp.arange(e2 - e, dtype=jnp.int32) % 128)])
    dum = n + (jnp.arange(NW * pad2, dtype=jnp.int32) % 128)
    src2 = jnp.concatenate(
        [src.reshape(NW, rpt), jnp.zeros((NW, pad2), jnp.int32)],
        axis=1).reshape(-1, 128)
    dst2 = jnp.concatenate(
        [dst.reshape(NW, rpt), dum.reshape(NW, pad2)],
        axis=1).reshape(-1, 128)
    zeros16 = jnp.zeros((na, LANES), jnp.float32)
    ones16 = jnp.ones((128, LANES), jnp.float32)

    deg0, deg1 = _sc_degree(dst2, zeros16, ones16, na=na, cpt=cpt)
    d0 = deg0.reshape(nr, PK)
    d1 = deg1.reshape(nr, PK)

    h1 = _tc_mm(x.T, W1.T, na)
    g1 = _tc_scale(h1.reshape(nr, PK), d0, d1, nr)

    p0, p1 = _sc_scatter(g1.reshape(na, LANES), src2, dst2, zeros16,
                         na=na, cpt=cpt)

    w2big = jnp.kron(jnp.eye(8, dtype=jnp.float32),
                     jnp.pad(W2, ((0, 0), (0, hid - nout))))
    b1r = jnp.tile(b1, 8).reshape(1, PK)
    g2 = _tc_mid(p0.reshape(nr, PK), p1.reshape(nr, PK), g1, d0, d1,
                 b1r, w2big, nr)

    q0, q1 = _sc_scatter(g2.reshape(na, LANES), src2, dst2, zeros16,
                         na=na, cpt=cpt)

    onesbig = jnp.kron(jnp.eye(8, dtype=jnp.float32),
                       jnp.ones((LANES, LANES), jnp.float32))
    b2r = jnp.tile(jnp.pad(b2, (0, hid - nout)), 8).reshape(1, PK)
    outpk = _tc_final(q0.reshape(nr, PK), q1.reshape(nr, PK), g2, d0, d1,
                      b2r, onesbig, nr, nout)
    return outpk.reshape(na, LANES)[:n, :nout]


# speedup vs baseline: 60.3401x; 1.1866x over previous
import functools

import jax
import jax.numpy as jnp
from jax import lax
from jax.experimental import pallas as pl
from jax.experimental.pallas import tpu as pltpu
from jax.experimental.pallas import tpu_sc as plsc

NC, NS, LANES = 2, 16, 16
NW = NC * NS
CH = 512
NRW = CH // 128
RB = 1024
PK = LANES * 8


def _mesh():
    return plsc.VectorSubcoreMesh(core_axis_name="c", subcore_axis_name="s")


_SC_PARAMS = pltpu.CompilerParams(use_tc_tiling_on_sc=False)


@functools.partial(jax.jit, static_argnames=("na", "cpt"))
def _sc_degree(dst2, zeros16, ones16, *, na, cpt):
    @functools.partial(
        pl.kernel,
        out_type=(jax.ShapeDtypeStruct((na, LANES), jnp.float32),
                  jax.ShapeDtypeStruct((na, LANES), jnp.float32)),
        mesh=_mesh(),
        scratch_types=[
            pltpu.VMEM((NRW, 128), jnp.int32),
            pltpu.VMEM((128, LANES), jnp.float32),
            pltpu.VMEM_SHARED((na, LANES), jnp.float32),
            pltpu.SemaphoreType.DMA,
        ],
        compiler_params=_SC_PARAMS,
    )
    def k(dst_hbm, zero_hbm, ones_hbm, out0, out1, dst_v, ones_v, acc, sem):
        cid = lax.axis_index("c")
        sid = lax.axis_index("s")
        wid = sid * NC + cid
        zpt = na // NS
        pltpu.sync_copy(zero_hbm.at[pl.ds(sid * zpt, zpt)],
                        acc.at[pl.ds(sid * zpt, zpt)])
        pltpu.sync_copy(ones_hbm, ones_v)
        plsc.subcore_barrier()

        def chunk(kk, carry):
            rbase = wid * (cpt * NRW) + kk * NRW
            pltpu.sync_copy(dst_hbm.at[pl.ds(rbase, NRW)], dst_v)
            hs = [pltpu.async_copy(ones_v, acc.at[dst_v.at[j]], sem, add=True)
                  for j in range(NRW)]
            for h in hs:
                h.wait()
            return carry

        lax.fori_loop(0, cpt, chunk, 0)
        plsc.subcore_barrier()

        @pl.when(cid == 0)
        def _():
            pltpu.sync_copy(acc.at[pl.ds(sid * zpt, zpt)],
                            out0.at[pl.ds(sid * zpt, zpt)])

        @pl.when(cid == 1)
        def _():
            pltpu.sync_copy(acc.at[pl.ds(sid * zpt, zpt)],
                            out1.at[pl.ds(sid * zpt, zpt)])

    return k(dst2, zeros16, ones16)


@functools.partial(jax.jit, static_argnames=("na", "cpt"))
def _sc_scatter(g, src2, dst2, zeros16, *, na, cpt):
    @functools.partial(
        pl.kernel,
        out_type=(jax.ShapeDtypeStruct((na, LANES), jnp.float32),
                  jax.ShapeDtypeStruct((na, LANES), jnp.float32)),
        mesh=_mesh(),
        scratch_types=[
            pltpu.VMEM((3, NRW, 128), jnp.int32),
            pltpu.VMEM((3, NRW, 128), jnp.int32),
            pltpu.VMEM((2, CH, LANES), jnp.float32),
            pltpu.VMEM_SHARED((na, LANES), jnp.float32),
            pltpu.VMEM_SHARED((na, LANES), jnp.float32),
            pltpu.SemaphoreType.DMA,
            pltpu.SemaphoreType.DMA,
            pltpu.SemaphoreType.DMA,
            pltpu.SemaphoreType.DMA,
            pltpu.SemaphoreType.DMA,
            pltpu.SemaphoreType.DMA,
        ],
        compiler_params=_SC_PARAMS,
    )
    def k(g_hbm, src_hbm, dst_hbm, zero_hbm, out0, out1,
          src_v, dst_v, rows_v, acc, gtab,
          semg, sems0, sems1, semi0, semi1, semi2):
        cid = lax.axis_index("c")
        sid = lax.axis_index("s")
        wid = sid * NC + cid
        zpt = na // NS
        pltpu.sync_copy(zero_hbm.at[pl.ds(sid * zpt, zpt)],
                        acc.at[pl.ds(sid * zpt, zpt)])
        pltpu.sync_copy(g_hbm.at[pl.ds(sid * zpt, zpt)],
                        gtab.at[pl.ds(sid * zpt, zpt)])
        plsc.subcore_barrier()

        sems = (sems0, sems1)
        semi = (semi0, semi1, semi2)
        pend = [None, None]
        ih = [None, None, None]

        def start_idx(kk):
            s = kk % 3
            rbase = wid * (cpt * NRW) + kk * NRW
            ih[s] = [
                pltpu.async_copy(src_hbm.at[pl.ds(rbase, NRW)],
                                 src_v.at[s], semi[s]),
                pltpu.async_copy(dst_hbm.at[pl.ds(rbase, NRW)],
                                 dst_v.at[s], semi[s]),
            ]

        start_idx(0)
        for kk in range(cpt):
            b = kk % 2
            s = kk % 3
            if pend[b] is not None:
                for h in pend[b]:
                    h.wait()
                pend[b] = None
            if kk + 1 < cpt:
                start_idx(kk + 1)
            for h in ih[s]:
                h.wait()
            gs = [pltpu.async_copy(gtab.at[src_v.at[s, j]],
                                   rows_v.at[b, pl.ds(j * 128, 128)], semg)
                  for j in range(NRW)]
            for h in gs:
                h.wait()
            pend[b] = [pltpu.async_copy(rows_v.at[b, pl.ds(j * 128, 128)],
                                        acc.at[dst_v.at[s, j]], sems[b],
                                        add=True)
                       for j in range(NRW)]
        for hs in pend:
            if hs is not None:
                for h in hs:
                    h.wait()
        plsc.subcore_barrier()

        @pl.when(cid == 0)
        def _():
            pltpu.sync_copy(acc.at[pl.ds(sid * zpt, zpt)],
                            out0.at[pl.ds(sid * zpt, zpt)])

        @pl.when(cid == 1)
        def _():
            pltpu.sync_copy(acc.at[pl.ds(sid * zpt, zpt)],
                            out1.at[pl.ds(sid * zpt, zpt)])

    return k(g, src2, dst2, zeros16)


def _mm_body(xt_ref, w1t_ref, h_ref):
    h_ref[...] = lax.dot_general(xt_ref[...], w1t_ref[...],
                                 (((0,), (1,)), ((), ())),
                                 preferred_element_type=jnp.float32)


def _scale_body(h_ref, d0_ref, d1_ref, g_ref):
    dinv = lax.rsqrt(d0_ref[...] + d1_ref[...] + 1.0)
    g_ref[...] = h_ref[...] * dinv


def _mid_body(p0_ref, p1_ref, g1_ref, d0_ref, d1_ref, b1_ref, w2_ref, o_ref):
    dinv = lax.rsqrt(d0_ref[...] + d1_ref[...] + 1.0)
    s = (p0_ref[...] + p1_ref[...] + g1_ref[...]) * dinv
    out1 = jnp.maximum(s + b1_ref[0, :][None, :], 0.0)
    h2 = jnp.dot(out1, w2_ref[...], preferred_element_type=jnp.float32)
    o_ref[...] = h2 * dinv


def _fin_body(nout, q0_ref, q1_ref, g2_ref, d0_ref, d1_ref, b2_ref, ones_ref,
              o_ref):
    dinv = lax.rsqrt(d0_ref[...] + d1_ref[...] + 1.0)
    z = (q0_ref[...] + q1_ref[...] + g2_ref[...]) * dinv + b2_ref[0, :][None, :]
    feat = lax.broadcasted_iota(jnp.int32, z.shape, 1) % LANES
    e = jnp.where(feat < nout, jnp.exp(z), 0.0)
    s = jnp.dot(e, ones_ref[...], preferred_element_type=jnp.float32)
    o_ref[...] = z - jnp.log(s)


PKR = 448


def _pk():
    return pl.BlockSpec((PKR, PK), lambda i: (i, 0))


def _row():
    return pl.BlockSpec((1, PK), lambda i: (0, 0))


def _sq():
    return pl.BlockSpec((PK, PK), lambda i: (0, 0))


def _tc_mm(xt, w1t, na):
    f = xt.shape[0]
    return pl.pallas_call(
        _mm_body,
        grid=(-(-na // RB),),
        in_specs=[
            pl.BlockSpec((f, RB), lambda i: (0, i)),
            pl.BlockSpec((LANES, f), lambda i: (0, 0)),
        ],
        out_specs=pl.BlockSpec((RB, LANES), lambda i: (i, 0)),
        out_shape=jax.ShapeDtypeStruct((na, LANES), jnp.float32),
    )(xt, w1t)


def _tc_scale(hpk, d0, d1, nr):
    return pl.pallas_call(
        _scale_body,
        grid=(-(-nr // PKR),),
        in_specs=[_pk(), _pk(), _pk()],
        out_specs=_pk(),
        out_shape=jax.ShapeDtypeStruct((nr, PK), jnp.float32),
    )(hpk, d0, d1)


def _tc_mid(p0, p1, g1, d0, d1, b1r, w2big, nr):
    return pl.pallas_call(
        _mid_body,
        grid=(-(-nr // PKR),),
        in_specs=[_pk(), _pk(), _pk(), _pk(), _pk(), _row(), _sq()],
        out_specs=_pk(),
        out_shape=jax.ShapeDtypeStruct((nr, PK), jnp.float32),
    )(p0, p1, g1, d0, d1, b1r, w2big)


def _tc_final(q0, q1, g2, d0, d1, b2r, onesbig, nr, nout):
    return pl.pallas_call(
        functools.partial(_fin_body, nout),
        grid=(-(-nr // PKR),),
        in_specs=[_pk(), _pk(), _pk(), _pk(), _pk(), _row(), _sq()],
        out_specs=_pk(),
        out_shape=jax.ShapeDtypeStruct((nr, PK), jnp.float32),
    )(q0, q1, g2, d0, d1, b2r, onesbig)


def kernel(x, edge_index, W1, b1, W2, b2):
    n, _ = x.shape
    e = edge_index.shape[1]
    hid = W1.shape[1]
    nout = W2.shape[1]
    assert hid == LANES

    cpt = -(-e // (NW * CH))
    ept = cpt * CH
    na = -(-(n + 128) // 128) * 128
    nr = na * LANES // PK

    e2 = -(-e // NW) * NW
    rpt = e2 // NW
    pad2 = ept - rpt
    src = edge_index[0]
    dst = edge_index[1]
    if e2 > e:
        src = jnp.concatenate([src, jnp.zeros((e2 - e,), jnp.int32)])
        dst = jnp.concatenate(
            [dst, n + (jnp.arange(e2 - e, dtype=jnp.int32) % 128)])
    dum = n + (jnp.arange(NW * pad2, dtype=jnp.int32) % 128)
    src2 = jnp.concatenate(
        [src.reshape(NW, rpt), jnp.zeros((NW, pad2), jnp.int32)],
        axis=1).reshape(-1, 128)
    dst2 = jnp.concatenate(
        [dst.reshape(NW, rpt), dum.reshape(NW, pad2)],
        axis=1).reshape(-1, 128)
    zeros16 = jnp.zeros((na, LANES), jnp.float32)
    ones16 = jnp.ones((128, LANES), jnp.float32)

    deg0, deg1 = _sc_degree(dst2, zeros16, ones16, na=na, cpt=cpt)
    d0 = deg0.reshape(nr, PK)
    d1 = deg1.reshape(nr, PK)

    h1 = _tc_mm(x.T, W1.T, na)
    g1 = _tc_scale(h1.reshape(nr, PK), d0, d1, nr)

    p0, p1 = _sc_scatter(g1.reshape(na, LANES), src2, dst2, zeros16,
                         na=na, cpt=cpt)

    w2big = jnp.kron(jnp.eye(8, dtype=jnp.float32),
                     jnp.pad(W2, ((0, 0), (0, hid - nout))))
    b1r = jnp.tile(b1, 8).reshape(1, PK)
    g2 = _tc_mid(p0.reshape(nr, PK), p1.reshape(nr, PK), g1, d0, d1,
                 b1r, w2big, nr)

    q0, q1 = _sc_scatter(g2.reshape(na, LANES), src2, dst2, zeros16,
                         na=na, cpt=cpt)

    onesbig = jnp.kron(jnp.eye(8, dtype=jnp.float32),
                       jnp.ones((LANES, LANES), jnp.float32))
    b2r = jnp.tile(jnp.pad(b2, (0, hid - nout)), 8).reshape(1, PK)
    outpk = _tc_final(q0.reshape(nr, PK), q1.reshape(nr, PK), g2, d0, d1,
                      b2r, onesbig, nr, nout)
    return outpk.reshape(na, LANES)[:n, :nout]


# speedup vs baseline: 63.0834x; 1.0455x over previous
import functools

import jax
import jax.numpy as jnp
from jax import lax
from jax.experimental import pallas as pl
from jax.experimental.pallas import tpu as pltpu
from jax.experimental.pallas import tpu_sc as plsc

NC, NS, LANES = 2, 16, 16
NW = NC * NS
CH = 512
NRW = CH // 128
RB = 1024
PK = LANES * 8


def _mesh():
    return plsc.VectorSubcoreMesh(core_axis_name="c", subcore_axis_name="s")


_SC_PARAMS = pltpu.CompilerParams(use_tc_tiling_on_sc=False)


@functools.partial(jax.jit, static_argnames=("na", "cpt"))
def _sc_degree(dst2, zeros16, ones16, *, na, cpt):
    @functools.partial(
        pl.kernel,
        out_type=(jax.ShapeDtypeStruct((na, LANES), jnp.float32),
                  jax.ShapeDtypeStruct((na, LANES), jnp.float32)),
        mesh=_mesh(),
        scratch_types=[
            pltpu.VMEM((NRW, 128), jnp.int32),
            pltpu.VMEM((128, LANES), jnp.float32),
            pltpu.VMEM_SHARED((na, LANES), jnp.float32),
            pltpu.SemaphoreType.DMA,
        ],
        compiler_params=_SC_PARAMS,
    )
    def k(dst_hbm, zero_hbm, ones_hbm, out0, out1, dst_v, ones_v, acc, sem):
        cid = lax.axis_index("c")
        sid = lax.axis_index("s")
        wid = sid * NC + cid
        zpt = na // NS
        pltpu.sync_copy(zero_hbm.at[pl.ds(sid * zpt, zpt)],
                        acc.at[pl.ds(sid * zpt, zpt)])
        pltpu.sync_copy(ones_hbm, ones_v)
        plsc.subcore_barrier()

        def chunk(kk, carry):
            rbase = wid * (cpt * NRW) + kk * NRW
            pltpu.sync_copy(dst_hbm.at[pl.ds(rbase, NRW)], dst_v)
            hs = [pltpu.async_copy(ones_v, acc.at[dst_v.at[j]], sem, add=True)
                  for j in range(NRW)]
            for h in hs:
                h.wait()
            return carry

        lax.fori_loop(0, cpt, chunk, 0)
        plsc.subcore_barrier()

        @pl.when(cid == 0)
        def _():
            pltpu.sync_copy(acc.at[pl.ds(sid * zpt, zpt)],
                            out0.at[pl.ds(sid * zpt, zpt)])

        @pl.when(cid == 1)
        def _():
            pltpu.sync_copy(acc.at[pl.ds(sid * zpt, zpt)],
                            out1.at[pl.ds(sid * zpt, zpt)])

    return k(dst2, zeros16, ones16)


@functools.partial(jax.jit, static_argnames=("na", "cpt"))
def _sc_scatter(g, src2, dst2, zeros16, *, na, cpt):
    @functools.partial(
        pl.kernel,
        out_type=(jax.ShapeDtypeStruct((na, LANES), jnp.float32),
                  jax.ShapeDtypeStruct((na, LANES), jnp.float32)),
        mesh=_mesh(),
        scratch_types=[
            pltpu.VMEM((3, NRW, 128), jnp.int32),
            pltpu.VMEM((3, NRW, 128), jnp.int32),
            pltpu.VMEM((2, CH, LANES), jnp.float32),
            pltpu.VMEM_SHARED((na, LANES), jnp.float32),
            pltpu.VMEM_SHARED((na, LANES), jnp.float32),
            pltpu.SemaphoreType.DMA,
            pltpu.SemaphoreType.DMA,
            pltpu.SemaphoreType.DMA,
            pltpu.SemaphoreType.DMA,
            pltpu.SemaphoreType.DMA,
            pltpu.SemaphoreType.DMA,
        ],
        compiler_params=_SC_PARAMS,
    )
    def k(g_hbm, src_hbm, dst_hbm, zero_hbm, out0, out1,
          src_v, dst_v, rows_v, acc, gtab,
          semg, sems0, sems1, semi0, semi1, semi2):
        cid = lax.axis_index("c")
        sid = lax.axis_index("s")
        wid = sid * NC + cid
        zpt = na // NS
        pltpu.sync_copy(zero_hbm.at[pl.ds(sid * zpt, zpt)],
                        acc.at[pl.ds(sid * zpt, zpt)])
        pltpu.sync_copy(g_hbm.at[pl.ds(sid * zpt, zpt)],
                        gtab.at[pl.ds(sid * zpt, zpt)])
        plsc.subcore_barrier()

        sems = (sems0, sems1)
        semi = (semi0, semi1, semi2)
        pend = [None, None]
        ih = [None, None, None]

        def start_idx(kk):
            s = kk % 3
            rbase = wid * (cpt * NRW) + kk * NRW
            ih[s] = [
                pltpu.async_copy(src_hbm.at[pl.ds(rbase, NRW)],
                                 src_v.at[s], semi[s]),
                pltpu.async_copy(dst_hbm.at[pl.ds(rbase, NRW)],
                                 dst_v.at[s], semi[s]),
            ]

        start_idx(0)
        for kk in range(cpt):
            b = kk % 2
            s = kk % 3
            if pend[b] is not None:
                for h in pend[b]:
                    h.wait()
                pend[b] = None
            if kk + 1 < cpt:
                start_idx(kk + 1)
            for h in ih[s]:
                h.wait()
            gs = [pltpu.async_copy(gtab.at[src_v.at[s, j]],
                                   rows_v.at[b, pl.ds(j * 128, 128)], semg)
                  for j in range(NRW)]
            for h in gs:
                h.wait()
            pend[b] = [pltpu.async_copy(rows_v.at[b, pl.ds(j * 128, 128)],
                                        acc.at[dst_v.at[s, j]], sems[b],
                                        add=True)
                       for j in range(NRW)]
        for hs in pend:
            if hs is not None:
                for h in hs:
                    h.wait()
        plsc.subcore_barrier()

        @pl.when(cid == 0)
        def _():
            pltpu.sync_copy(acc.at[pl.ds(sid * zpt, zpt)],
                            out0.at[pl.ds(sid * zpt, zpt)])

        @pl.when(cid == 1)
        def _():
            pltpu.sync_copy(acc.at[pl.ds(sid * zpt, zpt)],
                            out1.at[pl.ds(sid * zpt, zpt)])

    return k(g, src2, dst2, zeros16)


def _mm_body(xt_ref, w1t_ref, h_ref):
    h_ref[...] = lax.dot_general(xt_ref[...], w1t_ref[...],
                                 (((0,), (1,)), ((), ())),
                                 preferred_element_type=jnp.float32)


def _scale_body(h_ref, d0_ref, d1_ref, g_ref):
    dinv = lax.rsqrt(d0_ref[...] + d1_ref[...] + 1.0)
    g_ref[...] = h_ref[...] * dinv


def _mid_body(p0_ref, p1_ref, g1_ref, d0_ref, d1_ref, b1_ref, w2_ref, o_ref):
    dinv = lax.rsqrt(d0_ref[...] + d1_ref[...] + 1.0)
    s = (p0_ref[...] + p1_ref[...] + g1_ref[...]) * dinv
    out1 = jnp.maximum(s + b1_ref[0, :][None, :], 0.0)
    h2 = jnp.dot(out1, w2_ref[...], preferred_element_type=jnp.float32)
    o_ref[...] = h2 * dinv


def _fin_body(nout, q0_ref, q1_ref, g2_ref, d0_ref, d1_ref, b2_ref, ones_ref,
              o_ref):
    dinv = lax.rsqrt(d0_ref[...] + d1_ref[...] + 1.0)
    z = (q0_ref[...] + q1_ref[...] + g2_ref[...]) * dinv + b2_ref[0, :][None, :]
    feat = lax.broadcasted_iota(jnp.int32, z.shape, 1) % LANES
    e = jnp.where(feat < nout, jnp.exp(z), 0.0)
    s = jnp.dot(e, ones_ref[...], preferred_element_type=jnp.float32)
    o = z - jnp.log(s)
    o_ref[...] = jnp.transpose(o.reshape(PKR, 8, LANES),
                               (2, 0, 1)).reshape(LANES, PKR * 8)


PKR = 448


def _pk():
    return pl.BlockSpec((PKR, PK), lambda i: (i, 0))


def _row():
    return pl.BlockSpec((1, PK), lambda i: (0, 0))


def _sq():
    return pl.BlockSpec((PK, PK), lambda i: (0, 0))


def _tc_mm(xt, w1t, na):
    f = xt.shape[0]
    return pl.pallas_call(
        _mm_body,
        grid=(-(-na // RB),),
        in_specs=[
            pl.BlockSpec((f, RB), lambda i: (0, i)),
            pl.BlockSpec((LANES, f), lambda i: (0, 0)),
        ],
        out_specs=pl.BlockSpec((RB, LANES), lambda i: (i, 0)),
        out_shape=jax.ShapeDtypeStruct((na, LANES), jnp.float32),
    )(xt, w1t)


def _tc_scale(hpk, d0, d1, nr):
    return pl.pallas_call(
        _scale_body,
        grid=(-(-nr // PKR),),
        in_specs=[_pk(), _pk(), _pk()],
        out_specs=_pk(),
        out_shape=jax.ShapeDtypeStruct((nr, PK), jnp.float32),
    )(hpk, d0, d1)


def _tc_mid(p0, p1, g1, d0, d1, b1r, w2big, nr):
    return pl.pallas_call(
        _mid_body,
        grid=(-(-nr // PKR),),
        in_specs=[_pk(), _pk(), _pk(), _pk(), _pk(), _row(), _sq()],
        out_specs=_pk(),
        out_shape=jax.ShapeDtypeStruct((nr, PK), jnp.float32),
    )(p0, p1, g1, d0, d1, b1r, w2big)


def _tc_final(q0, q1, g2, d0, d1, b2r, onesbig, nr, nout):
    return pl.pallas_call(
        functools.partial(_fin_body, nout),
        grid=(-(-nr // PKR),),
        in_specs=[_pk(), _pk(), _pk(), _pk(), _pk(), _row(), _sq()],
        out_specs=pl.BlockSpec((LANES, PKR * 8), lambda i: (0, i)),
        out_shape=jax.ShapeDtypeStruct((LANES, nr * 8), jnp.float32),
    )(q0, q1, g2, d0, d1, b2r, onesbig)


def kernel(x, edge_index, W1, b1, W2, b2):
    n, _ = x.shape
    e = edge_index.shape[1]
    hid = W1.shape[1]
    nout = W2.shape[1]
    assert hid == LANES

    cpt = -(-e // (NW * CH))
    ept = cpt * CH
    na = -(-(n + 128) // 128) * 128
    nr = na * LANES // PK

    e2 = -(-e // NW) * NW
    rpt = e2 // NW
    pad2 = ept - rpt
    src = edge_index[0]
    dst = edge_index[1]
    if e2 > e:
        src = jnp.concatenate([src, jnp.zeros((e2 - e,), jnp.int32)])
        dst = jnp.concatenate(
            [dst, n + (jnp.arange(e2 - e, dtype=jnp.int32) % 128)])
    dum = n + (jnp.arange(NW * pad2, dtype=jnp.int32) % 128)
    src2 = jnp.concatenate(
        [src.reshape(NW, rpt), jnp.zeros((NW, pad2), jnp.int32)],
        axis=1).reshape(-1, 128)
    dst2 = jnp.concatenate(
        [dst.reshape(NW, rpt), dum.reshape(NW, pad2)],
        axis=1).reshape(-1, 128)
    zeros16 = jnp.zeros((na, LANES), jnp.float32)
    ones16 = jnp.ones((128, LANES), jnp.float32)

    deg0, deg1 = _sc_degree(dst2, zeros16, ones16, na=na, cpt=cpt)
    d0 = deg0.reshape(nr, PK)
    d1 = deg1.reshape(nr, PK)

    h1 = _tc_mm(x.T, W1.T, na)
    g1 = _tc_scale(h1.reshape(nr, PK), d0, d1, nr)

    p0, p1 = _sc_scatter(g1.reshape(na, LANES), src2, dst2, zeros16,
                         na=na, cpt=cpt)

    w2big = jnp.kron(jnp.eye(8, dtype=jnp.float32),
                     jnp.pad(W2, ((0, 0), (0, hid - nout))))
    b1r = jnp.tile(b1, 8).reshape(1, PK)
    g2 = _tc_mid(p0.reshape(nr, PK), p1.reshape(nr, PK), g1, d0, d1,
                 b1r, w2big, nr)

    q0, q1 = _sc_scatter(g2.reshape(na, LANES), src2, dst2, zeros16,
                         na=na, cpt=cpt)

    onesbig = jnp.kron(jnp.eye(8, dtype=jnp.float32),
                       jnp.ones((LANES, LANES), jnp.float32))
    b2r = jnp.tile(jnp.pad(b2, (0, hid - nout)), 8).reshape(1, PK)
    outt = _tc_final(q0.reshape(nr, PK), q1.reshape(nr, PK), g2, d0, d1,
                     b2r, onesbig, nr, nout)
    return outt[:nout, :n].T
